# baseline (device time: 99435 ns/iter reference)
import jax
import jax.numpy as jnp
from jax import lax
from jax.experimental import pallas as pl
from jax.experimental.pallas import tpu as pltpu

N_DEV = 32
H_R = 16
H_L = 15


def kernel(x, w_mat):
    m_per, k = x.shape
    _, n_per = w_mat.shape

    def body(
        x_ref,
        w_ref,
        out_ref,
        xb_ref,
        wb_ref,
        rbuf,
        lbuf,
        send_r,
        recv_r,
        send_l,
        recv_l,
    ):
        my = lax.axis_index("i")
        left = lax.rem(my + N_DEV - 1, N_DEV)
        right = lax.rem(my + 1, N_DEV)

        xb_ref[...] = x_ref[...].astype(jnp.bfloat16)
        wb_ref[...] = w_ref[...].astype(jnp.bfloat16)

        barrier = pltpu.get_barrier_semaphore()
        pl.semaphore_signal(
            barrier, inc=1, device_id=(left,), device_id_type=pl.DeviceIdType.MESH
        )
        pl.semaphore_signal(
            barrier, inc=1, device_id=(right,), device_id_type=pl.DeviceIdType.MESH
        )
        pl.semaphore_wait(barrier, 2)

        def make_r(h):
            src = xb_ref if h == 0 else rbuf.at[h - 1]
            return pltpu.make_async_remote_copy(
                src_ref=src,
                dst_ref=rbuf.at[h],
                send_sem=send_r.at[h],
                recv_sem=recv_r.at[h],
                device_id=(right,),
                device_id_type=pl.DeviceIdType.MESH,
            )

        def make_l(h):
            src = xb_ref if h == 0 else lbuf.at[h - 1]
            return pltpu.make_async_remote_copy(
                src_ref=src,
                dst_ref=lbuf.at[h],
                send_sem=send_l.at[h],
                recv_sem=recv_l.at[h],
                device_id=(left,),
                device_id_type=pl.DeviceIdType.MESH,
            )

        rd = [make_r(h) for h in range(H_R)]
        ld = [make_l(h) for h in range(H_L)]

        rd[0].start()
        ld[0].start()

        def gemm_store(chunk, origin):
            y = jnp.dot(chunk, wb_ref[...], preferred_element_type=jnp.float32)
            out_ref[pl.ds(origin * m_per, m_per), :] = jnp.maximum(y, 0.0)

        gemm_store(xb_ref[...], my)

        for h in range(H_R):
            rd[h].wait_recv()
            if h + 1 < H_R:
                rd[h + 1].start()
            if h < H_L:
                ld[h].wait_recv()
                if h + 1 < H_L:
                    ld[h + 1].start()
            gemm_store(rbuf[h], lax.rem(my + (N_DEV - 1 - h), N_DEV))
            if h < H_L:
                gemm_store(lbuf[h], lax.rem(my + 1 + h, N_DEV))

        for h in range(H_R):
            rd[h].wait_send()
        for h in range(H_L):
            ld[h].wait_send()

    return pl.pallas_call(
        body,
        out_shape=jax.ShapeDtypeStruct((N_DEV * m_per, n_per), jnp.float32),
        in_specs=[
            pl.BlockSpec(memory_space=pltpu.VMEM),
            pl.BlockSpec(memory_space=pltpu.VMEM),
        ],
        out_specs=pl.BlockSpec(memory_space=pltpu.VMEM),
        scratch_shapes=[
            pltpu.VMEM((m_per, k), jnp.bfloat16),
            pltpu.VMEM((k, n_per), jnp.bfloat16),
            pltpu.VMEM((H_R, m_per, k), jnp.bfloat16),
            pltpu.VMEM((H_L, m_per, k), jnp.bfloat16),
            pltpu.SemaphoreType.DMA((H_R,)),
            pltpu.SemaphoreType.DMA((H_R,)),
            pltpu.SemaphoreType.DMA((H_L,)),
            pltpu.SemaphoreType.DMA((H_L,)),
        ],
        compiler_params=pltpu.CompilerParams(collective_id=0),
    )(x, w_mat)


# device time: 83780 ns/iter; 1.1869x vs baseline; 1.1869x over previous
import jax
import jax.numpy as jnp
from jax import lax
from jax.experimental import pallas as pl
from jax.experimental.pallas import tpu as pltpu

N_DEV = 32
H_R = 16
H_L = 15

RING = [0, 3, 4, 7, 15, 12, 11, 8, 16, 19, 20, 23, 31, 28, 27, 24,
        25, 26, 29, 30, 22, 21, 18, 17, 9, 10, 13, 14, 6, 5, 2, 1]


def kernel(x, w_mat):
    m_per, k = x.shape
    _, n_per = w_mat.shape

    ring = jnp.array(RING, dtype=jnp.int32)
    inv = jnp.zeros(N_DEV, jnp.int32).at[ring].set(jnp.arange(N_DEV, dtype=jnp.int32))
    my = lax.axis_index("i")
    p = inv[my]
    left = ring[(p + N_DEV - 1) % N_DEV]
    right = ring[(p + 1) % N_DEV]
    orig_r = ring[(p + N_DEV - 1 - jnp.arange(H_R, dtype=jnp.int32)) % N_DEV]
    orig_l = ring[(p + 1 + jnp.arange(H_L, dtype=jnp.int32)) % N_DEV]
    meta = jnp.concatenate([jnp.stack([left, right]), orig_r, orig_l])

    def body(
        meta_ref,
        x_ref,
        w_ref,
        out_ref,
        xb_ref,
        wb_ref,
        rbuf,
        lbuf,
        send_r,
        recv_r,
        send_l,
        recv_l,
    ):
        my_id = lax.axis_index("i")
        left_t = meta_ref[0]
        right_t = meta_ref[1]

        xb_ref[...] = x_ref[...].astype(jnp.bfloat16)
        wb_ref[...] = w_ref[...].astype(jnp.bfloat16)

        barrier = pltpu.get_barrier_semaphore()
        pl.semaphore_signal(
            barrier, inc=1, device_id=(left_t,), device_id_type=pl.DeviceIdType.MESH
        )
        pl.semaphore_signal(
            barrier, inc=1, device_id=(right_t,), device_id_type=pl.DeviceIdType.MESH
        )
        pl.semaphore_wait(barrier, 2)

        def make_r(h):
            src = xb_ref if h == 0 else rbuf.at[h - 1]
            return pltpu.make_async_remote_copy(
                src_ref=src,
                dst_ref=rbuf.at[h],
                send_sem=send_r.at[h],
                recv_sem=recv_r.at[h],
                device_id=(right_t,),
                device_id_type=pl.DeviceIdType.MESH,
            )

        def make_l(h):
            src = xb_ref if h == 0 else lbuf.at[h - 1]
            return pltpu.make_async_remote_copy(
                src_ref=src,
                dst_ref=lbuf.at[h],
                send_sem=send_l.at[h],
                recv_sem=recv_l.at[h],
                device_id=(left_t,),
                device_id_type=pl.DeviceIdType.MESH,
            )

        rd = [make_r(h) for h in range(H_R)]
        ld = [make_l(h) for h in range(H_L)]

        rd[0].start()
        ld[0].start()

        def gemm_store(chunk, origin):
            y = jnp.dot(chunk, wb_ref[...], preferred_element_type=jnp.float32)
            out_ref[pl.ds(origin * m_per, m_per), :] = jnp.maximum(y, 0.0)

        gemm_store(xb_ref[...], my_id)

        for h in range(H_R):
            rd[h].wait_recv()
            if h + 1 < H_R:
                rd[h + 1].start()
            if h < H_L:
                ld[h].wait_recv()
                if h + 1 < H_L:
                    ld[h + 1].start()
            gemm_store(rbuf[h], meta_ref[2 + h])
            if h < H_L:
                gemm_store(lbuf[h], meta_ref[2 + H_R + h])

        for h in range(H_R):
            rd[h].wait_send()
        for h in range(H_L):
            ld[h].wait_send()

    return pl.pallas_call(
        body,
        out_shape=jax.ShapeDtypeStruct((N_DEV * m_per, n_per), jnp.float32),
        in_specs=[
            pl.BlockSpec(memory_space=pltpu.SMEM),
            pl.BlockSpec(memory_space=pltpu.VMEM),
            pl.BlockSpec(memory_space=pltpu.VMEM),
        ],
        out_specs=pl.BlockSpec(memory_space=pltpu.VMEM),
        scratch_shapes=[
            pltpu.VMEM((m_per, k), jnp.bfloat16),
            pltpu.VMEM((k, n_per), jnp.bfloat16),
            pltpu.VMEM((H_R, m_per, k), jnp.bfloat16),
            pltpu.VMEM((H_L, m_per, k), jnp.bfloat16),
            pltpu.SemaphoreType.DMA((H_R,)),
            pltpu.SemaphoreType.DMA((H_R,)),
            pltpu.SemaphoreType.DMA((H_L,)),
            pltpu.SemaphoreType.DMA((H_L,)),
        ],
        compiler_params=pltpu.CompilerParams(collective_id=0),
    )(meta, x, w_mat)


# device time: 61938 ns/iter; 1.6054x vs baseline; 1.3526x over previous
import jax
import jax.numpy as jnp
from jax import lax
from jax.experimental import pallas as pl
from jax.experimental.pallas import tpu as pltpu

N_DEV = 32
H_R = 16
H_L = 15
N_SUB = 4

RING = [0, 3, 4, 7, 15, 12, 11, 8, 16, 19, 20, 23, 31, 28, 27, 24,
        25, 26, 29, 30, 22, 21, 18, 17, 9, 10, 13, 14, 6, 5, 2, 1]


def kernel(x, w_mat):
    m_per, k = x.shape
    _, n_per = w_mat.shape

    ring = jnp.array(RING, dtype=jnp.int32)
    inv = jnp.zeros(N_DEV, jnp.int32).at[ring].set(jnp.arange(N_DEV, dtype=jnp.int32))
    my = lax.axis_index("i")
    p = inv[my]
    left = ring[(p + N_DEV - 1) % N_DEV]
    right = ring[(p + 1) % N_DEV]
    orig_r = ring[(p + N_DEV - 1 - jnp.arange(H_R, dtype=jnp.int32)) % N_DEV]
    orig_l = ring[(p + 1 + jnp.arange(H_L, dtype=jnp.int32)) % N_DEV]
    meta = jnp.concatenate([jnp.stack([left, right]), orig_r, orig_l])

    def body(
        meta_ref,
        x_ref,
        w_ref,
        out_ref,
        xb_ref,
        wb_ref,
        rbuf,
        lbuf,
        send_r,
        recv_r,
        send_l,
        recv_l,
    ):
        my_id = lax.axis_index("i")
        left_t = meta_ref[0]
        right_t = meta_ref[1]

        xb_ref[...] = x_ref[...].astype(jnp.bfloat16)
        wb_ref[...] = w_ref[...].astype(jnp.bfloat16)

        barrier = pltpu.get_barrier_semaphore()
        pl.semaphore_signal(
            barrier, inc=1, device_id=(left_t,), device_id_type=pl.DeviceIdType.MESH
        )
        pl.semaphore_signal(
            barrier, inc=1, device_id=(right_t,), device_id_type=pl.DeviceIdType.MESH
        )
        pl.semaphore_wait(barrier, 2)

        sub_m = m_per // N_SUB

        def make_r(h, s):
            rows = pl.ds(s * sub_m, sub_m)
            src = xb_ref.at[rows] if h == 0 else rbuf.at[h - 1, rows]
            return pltpu.make_async_remote_copy(
                src_ref=src,
                dst_ref=rbuf.at[h, rows],
                send_sem=send_r.at[h, s],
                recv_sem=recv_r.at[h, s],
                device_id=(right_t,),
                device_id_type=pl.DeviceIdType.MESH,
            )

        def make_l(h, s):
            rows = pl.ds(s * sub_m, sub_m)
            src = xb_ref.at[rows] if h == 0 else lbuf.at[h - 1, rows]
            return pltpu.make_async_remote_copy(
                src_ref=src,
                dst_ref=lbuf.at[h, rows],
                send_sem=send_l.at[h, s],
                recv_sem=recv_l.at[h, s],
                device_id=(left_t,),
                device_id_type=pl.DeviceIdType.MESH,
            )

        rd = [[make_r(h, s) for s in range(N_SUB)] for h in range(H_R)]
        ld = [[make_l(h, s) for s in range(N_SUB)] for h in range(H_L)]

        for s in range(N_SUB):
            rd[0][s].start()
        for s in range(N_SUB):
            ld[0][s].start()

        def gemm_store(chunk, origin):
            y = jnp.dot(chunk, wb_ref[...], preferred_element_type=jnp.float32)
            out_ref[pl.ds(origin * m_per, m_per), :] = jnp.maximum(y, 0.0)

        gemm_store(xb_ref[...], my_id)

        for h in range(H_R):
            for s in range(N_SUB):
                rd[h][s].wait_recv()
                if h + 1 < H_R:
                    rd[h + 1][s].start()
            if h < H_L:
                for s in range(N_SUB):
                    ld[h][s].wait_recv()
                    if h + 1 < H_L:
                        ld[h + 1][s].start()
            gemm_store(rbuf[h], meta_ref[2 + h])
            if h < H_L:
                gemm_store(lbuf[h], meta_ref[2 + H_R + h])

        for h in range(H_R):
            for s in range(N_SUB):
                rd[h][s].wait_send()
        for h in range(H_L):
            for s in range(N_SUB):
                ld[h][s].wait_send()

    return pl.pallas_call(
        body,
        out_shape=jax.ShapeDtypeStruct((N_DEV * m_per, n_per), jnp.float32),
        in_specs=[
            pl.BlockSpec(memory_space=pltpu.SMEM),
            pl.BlockSpec(memory_space=pltpu.VMEM),
            pl.BlockSpec(memory_space=pltpu.VMEM),
        ],
        out_specs=pl.BlockSpec(memory_space=pltpu.VMEM),
        scratch_shapes=[
            pltpu.VMEM((m_per, k), jnp.bfloat16),
            pltpu.VMEM((k, n_per), jnp.bfloat16),
            pltpu.VMEM((H_R, m_per, k), jnp.bfloat16),
            pltpu.VMEM((H_L, m_per, k), jnp.bfloat16),
            pltpu.SemaphoreType.DMA((H_R, N_SUB)),
            pltpu.SemaphoreType.DMA((H_R, N_SUB)),
            pltpu.SemaphoreType.DMA((H_L, N_SUB)),
            pltpu.SemaphoreType.DMA((H_L, N_SUB)),
        ],
        compiler_params=pltpu.CompilerParams(collective_id=0),
    )(meta, x, w_mat)
